# baseline (speedup 1.0000x reference)
import functools
import math

import jax
import jax.numpy as jnp
import numpy as np
from jax.experimental import pallas as pl
from jax.experimental.pallas import tpu as pltpu

NLAT = 61
NLON = 120
C = 256
NPIX = NLAT * NLON
NEG = -1e30
NLAT_P = 64
LP = 128
WINP = 3 * LP
NPIX_S = NLAT_P * LP
PSTEPS = 4
PCHUNK = 16 * NLON
ASTEPS = 16
RPS = NLAT_P // ASTEPS


@functools.lru_cache(maxsize=1)
def _mask_tables():
    theta = np.linspace(0.0, np.pi, NLAT)
    dtheta = np.pi / (NLAT - 1)
    w = np.sin(theta) * dtheta
    w[0] *= 0.5
    w[-1] *= 0.5
    w = np.maximum(w, 1e-4)
    quad_w = (2.0 * np.pi * w / NLON).astype(np.float64)
    log_qw = np.log(quad_w)

    phi = np.linspace(0.0, 2.0 * np.pi, NLON, endpoint=False)
    cut = (math.pi / (NLAT - 1)) * (1.0 + 1e-5)
    ct = np.cos(theta)[:, None]
    st = np.sin(theta)[:, None]
    cp = np.cos(phi)[None, :]

    lon = np.arange(NLON)
    cd = np.minimum(lon, NLON - lon)

    tbl = np.zeros((NLAT, 8), dtype=np.float32)
    for ho in range(NLAT):
        cosd = math.cos(theta[ho]) * ct + math.sin(theta[ho]) * st * cp
        dist = np.arccos(np.clip(cosd, -1.0, 1.0))
        hi, wi = np.nonzero(dist <= cut)
        base = min(max(ho - 1, 0), NLAT - 3)
        for r in range(3):
            sel = wi[hi == base + r]
            if len(sel) == 0:
                tbl[ho, r] = -1.0
                tbl[ho, 3 + r] = 0.0
                continue
            d = int(cd[sel].max())
            assert set(sel.tolist()) == {l for l in range(NLON) if cd[l] <= d}
            tbl[ho, r] = float(d)
            tbl[ho, 3 + r] = float(log_qw[base + r])

    dmap = np.full((LP, WINP), 1e9, dtype=np.float32)
    wo = np.arange(NLON)[:, None]
    delta = (np.arange(NLON)[None, :] - wo) % NLON
    cdm = np.minimum(delta, NLON - delta).astype(np.float32)
    for r in range(3):
        dmap[:NLON, r * LP:r * LP + NLON] = cdm
    return dmap, tbl


def _fused_kernel(tbl_ref, x_ref, w_ref, b_ref, dmap_ref, o_ref, qkv_ref):
    step = pl.program_id(0)

    @pl.when(step < PSTEPS)
    def _project():
        for p in range(PSTEPS):

            @pl.when(step == p)
            def _():
                lo = p * PCHUNK
                width = min(PCHUNK, NPIX - lo)
                xb = x_ref[:, lo:lo + width].astype(jnp.bfloat16)
                qkv = (
                    jax.lax.dot_general(
                        xb, w_ref[...], (((0,), (0,)), ((), ())),
                        preferred_element_type=jnp.float32,
                    )
                    + b_ref[...]
                ).astype(jnp.bfloat16)
                for j in range(width // NLON):
                    row = (p * 16 + j) * LP
                    qkv_ref[row:row + NLON, :] = (
                        qkv[j * NLON:(j + 1) * NLON, :]
                    )
                    qkv_ref[row + NLON:row + LP, :] = jnp.zeros(
                        (LP - NLON, 3 * C), jnp.bfloat16
                    )

    @pl.when(step == PSTEPS - 1)
    def _zero_tail():
        qkv_ref[NLAT * LP:, :] = jnp.zeros(
            ((NLAT_P - NLAT) * LP, 3 * C), jnp.bfloat16
        )

    @pl.when(step >= PSTEPS)
    def _attend():
        g = step - PSTEPS
        col = jax.lax.broadcasted_iota(jnp.int32, (1, WINP), 1)
        dmap = dmap_ref[...]

        def band(v0, v1, v2):
            return jnp.where(col < LP, v0, jnp.where(col < 2 * LP, v1, v2))

        for r in range(RPS):
            ho = g * RPS + r
            hoc = jnp.minimum(ho, NLAT - 1)
            base = jnp.clip(ho - 1, 0, NLAT - 3) * LP
            k_win = qkv_ref[pl.ds(base, WINP), C:2 * C]
            v_win = qkv_ref[pl.ds(base, WINP), 2 * C:]
            q = qkv_ref[pl.ds(ho * LP, LP), :C]
            corr = jax.lax.dot_general(
                q, k_win, (((1,), (1,)), ((), ())),
                preferred_element_type=jnp.float32,
            )
            dvec = band(tbl_ref[hoc, 0], tbl_ref[hoc, 1], tbl_ref[hoc, 2])
            lqvec = band(tbl_ref[hoc, 3], tbl_ref[hoc, 4], tbl_ref[hoc, 5])
            s = jnp.where(dmap <= dvec, corr + lqvec, NEG)
            m = jnp.max(s, axis=1, keepdims=True)
            a = jnp.exp(s - m)
            attn = (a / jnp.sum(a, axis=1, keepdims=True)).astype(jnp.bfloat16)
            o_ref[r * LP:(r + 1) * LP, :] = jax.lax.dot_general(
                attn, v_win, (((1,), (0,)), ((), ())),
                preferred_element_type=jnp.float32,
            )


def kernel(query, q_weights, k_weights, v_weights, q_bias, k_bias, v_bias):
    scale = math.sqrt(1.0 / C)
    x = query[0].reshape(C, NPIX)
    w_cat = jnp.concatenate(
        [(scale * q_weights).T, k_weights.T, v_weights.T], axis=1
    ).astype(jnp.bfloat16)
    b_cat = jnp.concatenate([q_bias, k_bias, v_bias]).reshape(1, 3 * C)

    dmap_np, tbl_np = _mask_tables()
    out = pl.pallas_call(
        _fused_kernel,
        grid=(PSTEPS + ASTEPS,),
        in_specs=[
            pl.BlockSpec(memory_space=pltpu.SMEM),
            pl.BlockSpec((C, NPIX), lambda h: (0, 0)),
            pl.BlockSpec((C, 3 * C), lambda h: (0, 0)),
            pl.BlockSpec((1, 3 * C), lambda h: (0, 0)),
            pl.BlockSpec((LP, WINP), lambda h: (0, 0)),
        ],
        out_specs=pl.BlockSpec(
            (RPS * LP, C), lambda h: (jnp.maximum(h - PSTEPS, 0), 0)
        ),
        out_shape=jax.ShapeDtypeStruct((NPIX_S, C), jnp.float32),
        scratch_shapes=[pltpu.VMEM((NPIX_S, 3 * C), jnp.bfloat16)],
    )(jnp.asarray(tbl_np), x, w_cat, b_cat, jnp.asarray(dmap_np))

    return (
        out.reshape(NLAT_P, LP, C)[:NLAT, :NLON]
        .transpose(2, 0, 1)
        .reshape(1, C, NLAT, NLON)
    )


# speedup vs baseline: 1.0443x; 1.0443x over previous
import functools
import math

import jax
import jax.numpy as jnp
import numpy as np
from jax.experimental import pallas as pl
from jax.experimental.pallas import tpu as pltpu

NLAT = 61
NLON = 120
C = 256
NPIX = NLAT * NLON
NEG = -1e30
NLAT_P = 64
LP = 128
WINP = 3 * LP
NPIX_S = NLAT_P * LP
PSTEPS = 4
PCHUNK = 16 * NLON
ASTEPS = 4
RPS = NLAT_P // ASTEPS


@functools.lru_cache(maxsize=1)
def _mask_tables():
    theta = np.linspace(0.0, np.pi, NLAT)
    dtheta = np.pi / (NLAT - 1)
    w = np.sin(theta) * dtheta
    w[0] *= 0.5
    w[-1] *= 0.5
    w = np.maximum(w, 1e-4)
    quad_w = (2.0 * np.pi * w / NLON).astype(np.float64)
    log_qw = np.log(quad_w)

    phi = np.linspace(0.0, 2.0 * np.pi, NLON, endpoint=False)
    cut = (math.pi / (NLAT - 1)) * (1.0 + 1e-5)
    ct = np.cos(theta)[:, None]
    st = np.sin(theta)[:, None]
    cp = np.cos(phi)[None, :]

    lon = np.arange(NLON)
    cd = np.minimum(lon, NLON - lon)

    tbl = np.zeros((NLAT, 8), dtype=np.float32)
    for ho in range(NLAT):
        cosd = math.cos(theta[ho]) * ct + math.sin(theta[ho]) * st * cp
        dist = np.arccos(np.clip(cosd, -1.0, 1.0))
        hi, wi = np.nonzero(dist <= cut)
        base = min(max(ho - 1, 0), NLAT - 3)
        for r in range(3):
            sel = wi[hi == base + r]
            if len(sel) == 0:
                tbl[ho, r] = -1.0
                tbl[ho, 3 + r] = 0.0
                continue
            d = int(cd[sel].max())
            assert set(sel.tolist()) == {l for l in range(NLON) if cd[l] <= d}
            tbl[ho, r] = float(d)
            tbl[ho, 3 + r] = float(log_qw[base + r])

    dmap = np.full((LP, WINP), 1e9, dtype=np.float32)
    wo = np.arange(NLON)[:, None]
    delta = (np.arange(NLON)[None, :] - wo) % NLON
    cdm = np.minimum(delta, NLON - delta).astype(np.float32)
    for r in range(3):
        dmap[:NLON, r * LP:r * LP + NLON] = cdm
    return dmap, tbl


def _fused_kernel(tbl_ref, x_ref, w_ref, b_ref, dmap_ref, o_ref, qkv_ref):
    step = pl.program_id(0)

    @pl.when(step < PSTEPS)
    def _project():
        for p in range(PSTEPS):

            @pl.when(step == p)
            def _():
                lo = p * PCHUNK
                width = min(PCHUNK, NPIX - lo)
                xb = x_ref[:, lo:lo + width].astype(jnp.bfloat16)
                qkv = (
                    jax.lax.dot_general(
                        xb, w_ref[...], (((0,), (0,)), ((), ())),
                        preferred_element_type=jnp.float32,
                    )
                    + b_ref[...]
                ).astype(jnp.bfloat16)
                for j in range(width // NLON):
                    row = (p * 16 + j) * LP
                    qkv_ref[row:row + NLON, :] = (
                        qkv[j * NLON:(j + 1) * NLON, :]
                    )
                    qkv_ref[row + NLON:row + LP, :] = jnp.zeros(
                        (LP - NLON, 3 * C), jnp.bfloat16
                    )

    @pl.when(step == PSTEPS - 1)
    def _zero_tail():
        qkv_ref[NLAT * LP:, :] = jnp.zeros(
            ((NLAT_P - NLAT) * LP, 3 * C), jnp.bfloat16
        )

    @pl.when(step >= PSTEPS)
    def _attend():
        g = step - PSTEPS
        col = jax.lax.broadcasted_iota(jnp.int32, (1, WINP), 1)
        dmap = dmap_ref[...]

        def band(v0, v1, v2):
            return jnp.where(col < LP, v0, jnp.where(col < 2 * LP, v1, v2))

        for r in range(RPS):
            ho = g * RPS + r
            hoc = jnp.minimum(ho, NLAT - 1)
            base = jnp.clip(ho - 1, 0, NLAT - 3) * LP
            k_win = qkv_ref[pl.ds(base, WINP), C:2 * C]
            v_win = qkv_ref[pl.ds(base, WINP), 2 * C:]
            q = qkv_ref[pl.ds(ho * LP, LP), :C]
            corr = jax.lax.dot_general(
                q, k_win, (((1,), (1,)), ((), ())),
                preferred_element_type=jnp.float32,
            )
            dvec = band(tbl_ref[hoc, 0], tbl_ref[hoc, 1], tbl_ref[hoc, 2])
            lqvec = band(tbl_ref[hoc, 3], tbl_ref[hoc, 4], tbl_ref[hoc, 5])
            s = jnp.where(dmap <= dvec, corr + lqvec, NEG)
            m = jnp.max(s, axis=1, keepdims=True)
            a = jnp.exp(s - m)
            attn = (a / jnp.sum(a, axis=1, keepdims=True)).astype(jnp.bfloat16)
            o_ref[r * LP:(r + 1) * LP, :] = jax.lax.dot_general(
                attn, v_win, (((1,), (0,)), ((), ())),
                preferred_element_type=jnp.float32,
            )


def kernel(query, q_weights, k_weights, v_weights, q_bias, k_bias, v_bias):
    scale = math.sqrt(1.0 / C)
    x = query[0].reshape(C, NPIX)
    w_cat = jnp.concatenate(
        [(scale * q_weights).T, k_weights.T, v_weights.T], axis=1
    ).astype(jnp.bfloat16)
    b_cat = jnp.concatenate([q_bias, k_bias, v_bias]).reshape(1, 3 * C)

    dmap_np, tbl_np = _mask_tables()
    out = pl.pallas_call(
        _fused_kernel,
        grid=(PSTEPS + ASTEPS,),
        in_specs=[
            pl.BlockSpec(memory_space=pltpu.SMEM),
            pl.BlockSpec((C, NPIX), lambda h: (0, 0)),
            pl.BlockSpec((C, 3 * C), lambda h: (0, 0)),
            pl.BlockSpec((1, 3 * C), lambda h: (0, 0)),
            pl.BlockSpec((LP, WINP), lambda h: (0, 0)),
        ],
        out_specs=pl.BlockSpec(
            (RPS * LP, C), lambda h: (jnp.maximum(h - PSTEPS, 0), 0)
        ),
        out_shape=jax.ShapeDtypeStruct((NPIX_S, C), jnp.float32),
        scratch_shapes=[pltpu.VMEM((NPIX_S, 3 * C), jnp.bfloat16)],
    )(jnp.asarray(tbl_np), x, w_cat, b_cat, jnp.asarray(dmap_np))

    return (
        out.reshape(NLAT_P, LP, C)[:NLAT, :NLON]
        .transpose(2, 0, 1)
        .reshape(1, C, NLAT, NLON)
    )
